# baseline (speedup 1.0000x reference)
import functools

import jax
import jax.numpy as jnp
from jax.experimental import pallas as pl
from jax.experimental.pallas import tpu as pltpu

_NUM_ADAPTERS = 8
_MAX_RANK = 64
_ER = _NUM_ADAPTERS * _MAX_RANK
_TB = 512


def _lora_kernel(ids_ref, ranks_ref, x_ref, a_ref, b_ref, w_ref, out_ref,
                 bt_ref):
    @pl.when(pl.program_id(0) == 0)
    def _():
        for e in range(_NUM_ADAPTERS):
            bt_ref[e * _MAX_RANK:(e + 1) * _MAX_RANK, :] = jnp.transpose(
                b_ref[e], (1, 0))

    x = x_ref[...]
    base = jax.lax.dot_general(
        x, w_ref[...], (((1,), (1,)), ((), ())),
        preferred_element_type=jnp.float32)

    xa = jax.lax.dot_general(
        x, a_ref[...], (((1,), (1,)), ((), ())),
        preferred_element_type=jnp.float32)

    ids = ids_ref[0]
    col = jax.lax.broadcasted_iota(jnp.int32, (1, _ER), 1)
    col_e = col // _MAX_RANK
    col_r = col - col_e * _MAX_RANK
    col_rank = jnp.zeros((1, _ER), jnp.int32)
    for e in range(_NUM_ADAPTERS):
        col_rank = jnp.where(col_e == e, ranks_ref[e], col_rank)
    mask = jnp.logical_and(col_e == ids, col_r < col_rank).astype(jnp.float32)
    xa = xa * mask

    lora = jax.lax.dot_general(
        xa, bt_ref[...], (((1,), (0,)), ((), ())),
        preferred_element_type=jnp.float32)
    out_ref[...] = base + lora


@functools.partial(jax.jit, static_argnames=())
def kernel(x, adapter_ids, ranks, a_cache, b_cache, W):
    tok, d_in = x.shape
    d_out = W.shape[0]
    nb = tok // _TB
    ids = adapter_ids.astype(jnp.int32).reshape(nb, _TB, 1)
    ranks32 = ranks.astype(jnp.int32)
    a_stack = a_cache.reshape(_ER, d_in)

    grid_spec = pltpu.PrefetchScalarGridSpec(
        num_scalar_prefetch=0,
        grid=(nb,),
        in_specs=[
            pl.BlockSpec((1, _TB, 1), lambda i: (i, 0, 0)),
            pl.BlockSpec(memory_space=pltpu.SMEM),
            pl.BlockSpec((_TB, d_in), lambda i: (i, 0)),
            pl.BlockSpec((_ER, d_in), lambda i: (0, 0)),
            pl.BlockSpec((_NUM_ADAPTERS, d_out, _MAX_RANK), lambda i: (0, 0, 0)),
            pl.BlockSpec((d_out, d_in), lambda i: (0, 0)),
        ],
        out_specs=pl.BlockSpec((_TB, d_out), lambda i: (i, 0)),
        scratch_shapes=[pltpu.VMEM((_ER, d_out), jnp.float32)],
    )

    out = pl.pallas_call(
        _lora_kernel,
        grid_spec=grid_spec,
        out_shape=jax.ShapeDtypeStruct((tok, d_out), jnp.float32),
        compiler_params=pltpu.CompilerParams(
            dimension_semantics=("arbitrary",),
        ),
    )(ids, ranks32, x, a_stack, b_cache, W)
    return out


# speedup vs baseline: 1.0982x; 1.0982x over previous
import functools

import jax
import jax.numpy as jnp
from jax.experimental import pallas as pl
from jax.experimental.pallas import tpu as pltpu

_NUM_ADAPTERS = 8
_MAX_RANK = 64
_ER = _NUM_ADAPTERS * _MAX_RANK
_TB = 512


def _lora_kernel(ids_ref, ranks_ref, x_ref, a_ref, bt_ref, w_ref, out_ref):
    x = x_ref[...]
    base = jax.lax.dot_general(
        x, w_ref[...], (((1,), (1,)), ((), ())),
        preferred_element_type=jnp.float32)

    xa = jax.lax.dot_general(
        x, a_ref[...], (((1,), (1,)), ((), ())),
        preferred_element_type=jnp.float32)

    ids = ids_ref[0]
    col = jax.lax.broadcasted_iota(jnp.int32, (1, _ER), 1)
    col_e = col // _MAX_RANK
    col_r = col - col_e * _MAX_RANK
    col_rank = jnp.zeros((1, _ER), jnp.int32)
    for e in range(_NUM_ADAPTERS):
        col_rank = jnp.where(col_e == e, ranks_ref[e], col_rank)
    mask = jnp.logical_and(col_e == ids, col_r < col_rank).astype(jnp.float32)
    xa = xa * mask

    lora = jax.lax.dot_general(
        xa, bt_ref[...], (((1,), (0,)), ((), ())),
        preferred_element_type=jnp.float32)
    out_ref[...] = base + lora


@functools.partial(jax.jit, static_argnames=())
def kernel(x, adapter_ids, ranks, a_cache, b_cache, W):
    tok, d_in = x.shape
    d_out = W.shape[0]
    nb = tok // _TB
    ids = adapter_ids.astype(jnp.int32).reshape(nb, _TB, 1)
    ranks32 = ranks.astype(jnp.int32)
    a_stack = a_cache.reshape(_ER, d_in)
    bt_stack = b_cache.transpose(0, 2, 1).reshape(_ER, d_out)

    grid_spec = pltpu.PrefetchScalarGridSpec(
        num_scalar_prefetch=0,
        grid=(nb,),
        in_specs=[
            pl.BlockSpec((1, _TB, 1), lambda i: (i, 0, 0)),
            pl.BlockSpec(memory_space=pltpu.SMEM),
            pl.BlockSpec((_TB, d_in), lambda i: (i, 0)),
            pl.BlockSpec((_ER, d_in), lambda i: (0, 0)),
            pl.BlockSpec((_ER, d_out), lambda i: (0, 0)),
            pl.BlockSpec((d_out, d_in), lambda i: (0, 0)),
        ],
        out_specs=pl.BlockSpec((_TB, d_out), lambda i: (i, 0)),
    )

    out = pl.pallas_call(
        _lora_kernel,
        grid_spec=grid_spec,
        out_shape=jax.ShapeDtypeStruct((tok, d_out), jnp.float32),
        compiler_params=pltpu.CompilerParams(
            dimension_semantics=("arbitrary",),
        ),
    )(ids, ranks32, x, a_stack, bt_stack, W)
    return out
